# Initial kernel scaffold; baseline (speedup 1.0000x reference)
import jax
import jax.numpy as jnp
from jax.experimental import pallas as pl


def kernel(x, edge_index, fouriercoeffs, bias):
    raise NotImplementedError("write your pallas kernel here")


# speedup vs baseline: 4.0129x; 4.0129x over previous
import functools

import jax
import jax.numpy as jnp
from jax import lax
from jax.experimental import pallas as pl
from jax.experimental.pallas import tpu as pltpu
from jax.experimental.pallas import tpu_sc as plsc

NC = 2
NS = 16


def _fourier_msg(x, w2, grid_size):
    n, in_feats = x.shape
    two_gi = w2.shape[0]
    out_feats = w2.shape[1]
    bn = 400
    assert n % bn == 0

    def body(x_ref, w_ref, o_ref, feats_ref):
        xb = x_ref[...]
        for k in range(grid_size):
            kf = jnp.float32(k + 1)
            feats_ref[:, k * in_feats:(k + 1) * in_feats] = jnp.cos(kf * xb)
            feats_ref[:, (grid_size + k) * in_feats:(grid_size + k + 1) * in_feats] = (
                jnp.sin(kf * xb))
        o_ref[...] = jnp.dot(feats_ref[...], w_ref[...],
                             preferred_element_type=jnp.float32)

    return pl.pallas_call(
        body,
        grid=(n // bn,),
        in_specs=[
            pl.BlockSpec((bn, in_feats), lambda i: (i, 0)),
            pl.BlockSpec((two_gi, out_feats), lambda i: (0, 0)),
        ],
        out_specs=pl.BlockSpec((bn, out_feats), lambda i: (i, 0)),
        out_shape=jax.ShapeDtypeStruct((n, out_feats), jnp.float32),
        scratch_shapes=[pltpu.VMEM((bn, two_gi), jnp.float32)],
    )(x, w2)


def _edge_scatter(msg, src, dst, zeros_blk):
    n, out_feats = msg.shape
    e = src.shape[0]
    epw = e // (NC * NS)
    chunk = 80
    assert epw % chunk == 0
    row_stride = (n // NS) // 8 * 8
    row_win = n - (NS - 1) * row_stride
    assert row_win >= row_stride and row_win % 8 == 0

    mesh = plsc.VectorSubcoreMesh(core_axis_name="c", subcore_axis_name="s")

    @functools.partial(
        pl.kernel,
        out_type=jax.ShapeDtypeStruct((NC, n, out_feats), jnp.float32),
        mesh=mesh,
        scratch_types=[
            pltpu.VMEM((chunk,), jnp.int32),
            pltpu.VMEM((chunk,), jnp.int32),
            pltpu.VMEM((chunk, out_feats), jnp.float32),
            pltpu.VMEM_SHARED((n, out_feats), jnp.float32),
        ],
    )
    def k(msg_hbm, src_hbm, dst_hbm, zero_hbm, out_hbm, sidx, didx, rows, acc):
        c = lax.axis_index("c")
        s = lax.axis_index("s")
        base = (c * NS + s) * epw

        pltpu.sync_copy(zero_hbm, acc.at[pl.ds(s * row_stride, row_win)])
        plsc.subcore_barrier()

        @pl.loop(0, epw, step=chunk)
        def _(i):
            off = base + i
            pltpu.sync_copy(src_hbm.at[pl.ds(off, chunk)], sidx)
            pltpu.sync_copy(dst_hbm.at[pl.ds(off, chunk)], didx)
            pltpu.sync_copy(msg_hbm.at[sidx], rows)
            pltpu.sync_copy(rows, acc.at[didx], add=True)

        plsc.subcore_barrier()
        pltpu.sync_copy(acc.at[pl.ds(s * row_stride, row_win)],
                        out_hbm.at[c].at[pl.ds(s * row_stride, row_win)])

    return k(msg, src, dst, zeros_blk)


def _combine(parts, bias2d):
    _, n, out_feats = parts.shape
    bn = 1000
    assert n % bn == 0

    def body(p_ref, b_ref, o_ref):
        o_ref[...] = p_ref[0] + p_ref[1] + b_ref[...]

    return pl.pallas_call(
        body,
        grid=(n // bn,),
        in_specs=[
            pl.BlockSpec((NC, bn, out_feats), lambda i: (0, i, 0)),
            pl.BlockSpec((1, out_feats), lambda i: (0, 0)),
        ],
        out_specs=pl.BlockSpec((bn, out_feats), lambda i: (i, 0)),
        out_shape=jax.ShapeDtypeStruct((n, out_feats), jnp.float32),
    )(parts, bias2d)


def kernel(x, edge_index, fouriercoeffs, bias):
    n, in_feats = x.shape
    out_feats = fouriercoeffs.shape[1]
    grid_size = fouriercoeffs.shape[3]
    w2 = jnp.transpose(fouriercoeffs, (0, 3, 2, 1)).reshape(
        2 * grid_size * in_feats, out_feats)
    msg = _fourier_msg(x, w2, grid_size)
    src = edge_index[0]
    dst = edge_index[1]
    row_win = n - (NS - 1) * ((n // NS) // 8 * 8)
    zeros_blk = jnp.zeros((row_win, out_feats), jnp.float32)
    parts = _edge_scatter(msg, src, dst, zeros_blk)
    return _combine(parts, bias.reshape(1, out_feats))


# speedup vs baseline: 6.5609x; 1.6350x over previous
import functools

import jax
import jax.numpy as jnp
from jax import lax
from jax.experimental import pallas as pl
from jax.experimental.pallas import tpu as pltpu
from jax.experimental.pallas import tpu_sc as plsc

NC = 2
NS = 16
CH = 128


def _fourier_msg(x, w2, grid_size):
    n, in_feats = x.shape
    two_gi = w2.shape[0]
    out_feats = w2.shape[1]
    bn = 400
    assert n % bn == 0

    def body(x_ref, w_ref, o_ref, feats_ref):
        xb = x_ref[...]
        for k in range(grid_size):
            kf = jnp.float32(k + 1)
            feats_ref[:, k * in_feats:(k + 1) * in_feats] = (
                jnp.cos(kf * xb).astype(jnp.bfloat16))
            feats_ref[:, (grid_size + k) * in_feats:(grid_size + k + 1) * in_feats] = (
                jnp.sin(kf * xb).astype(jnp.bfloat16))
        o_ref[...] = jnp.dot(feats_ref[...], w_ref[...],
                             preferred_element_type=jnp.float32)

    return pl.pallas_call(
        body,
        grid=(n // bn,),
        in_specs=[
            pl.BlockSpec((bn, in_feats), lambda i: (i, 0)),
            pl.BlockSpec((two_gi, out_feats), lambda i: (0, 0)),
        ],
        out_specs=pl.BlockSpec((bn, out_feats), lambda i: (i, 0)),
        out_shape=jax.ShapeDtypeStruct((n, out_feats), jnp.float32),
        scratch_shapes=[pltpu.VMEM((bn, two_gi), jnp.bfloat16)],
    )(x, w2)


def _edge_scatter(msg, pidx3, zeros_blk):
    n, out_feats = msg.shape
    nchunks = pidx3.shape[0]
    nw = NC * NS
    full_rounds = nchunks // nw
    extra = nchunks - full_rounds * nw
    npairs = full_rounds // 2
    assert full_rounds % 2 == 0
    row_stride = (n // NS) // 8 * 8
    row_win = n - (NS - 1) * row_stride
    assert row_win >= row_stride and row_win % 8 == 0

    mesh = plsc.VectorSubcoreMesh(core_axis_name="c", subcore_axis_name="s")

    @functools.partial(
        pl.kernel,
        out_type=jax.ShapeDtypeStruct((NC, n, out_feats), jnp.float32),
        mesh=mesh,
        scratch_types=[
            pltpu.VMEM((2, CH), jnp.int32),
            pltpu.VMEM((2, CH), jnp.int32),
            pltpu.VMEM((CH, out_feats), jnp.float32),
            pltpu.VMEM((CH, out_feats), jnp.float32),
            pltpu.VMEM_SHARED((n, out_feats), jnp.float32),
            pltpu.SemaphoreType.DMA,
            pltpu.SemaphoreType.DMA,
        ],
    )
    def k(msg_hbm, p3_hbm, zero_hbm, out_hbm,
          pidx0, pidx1, rows0, rows1, acc, sem0, sem1):
        c = lax.axis_index("c")
        s = lax.axis_index("s")
        w = c * NS + s

        pltpu.sync_copy(zero_hbm, acc.at[pl.ds(s * row_stride, row_win)])
        plsc.subcore_barrier()

        pltpu.sync_copy(p3_hbm.at[w], pidx0)
        pltpu.async_copy(msg_hbm.at[pidx0.at[0]], rows0, sem0)

        @pl.loop(0, npairs)
        def _(p):
            ga = w + nw * (2 * p)
            gb = ga + nw
            pltpu.sync_copy(p3_hbm.at[gb], pidx1)
            cb = pltpu.async_copy(msg_hbm.at[pidx1.at[0]], rows1, sem1)
            pltpu.make_async_copy(msg_hbm.at[pidx0.at[0]], rows0, sem0).wait()
            pltpu.sync_copy(rows0, acc.at[pidx0.at[1]], add=True)
            @pl.when(p < npairs - 1)
            def _():
                pltpu.sync_copy(p3_hbm.at[gb + nw], pidx0)
                pltpu.async_copy(msg_hbm.at[pidx0.at[0]], rows0, sem0)
            cb.wait()
            pltpu.sync_copy(rows1, acc.at[pidx1.at[1]], add=True)

        @pl.when(w < extra)
        def _():
            pltpu.sync_copy(p3_hbm.at[full_rounds * nw + w], pidx0)
            pltpu.async_copy(msg_hbm.at[pidx0.at[0]], rows0, sem0).wait()
            pltpu.sync_copy(rows0, acc.at[pidx0.at[1]], add=True)

        plsc.subcore_barrier()
        pltpu.sync_copy(acc.at[pl.ds(s * row_stride, row_win)],
                        out_hbm.at[c].at[pl.ds(s * row_stride, row_win)])

    return k(msg, pidx3, zeros_blk)


def _combine(parts, bias2d):
    _, n, out_feats = parts.shape
    bn = 1000
    assert n % bn == 0

    def body(p_ref, b_ref, o_ref):
        o_ref[...] = p_ref[0] + p_ref[1] + b_ref[...]

    return pl.pallas_call(
        body,
        grid=(n // bn,),
        in_specs=[
            pl.BlockSpec((NC, bn, out_feats), lambda i: (0, i, 0)),
            pl.BlockSpec((1, out_feats), lambda i: (0, 0)),
        ],
        out_specs=pl.BlockSpec((bn, out_feats), lambda i: (i, 0)),
        out_shape=jax.ShapeDtypeStruct((n, out_feats), jnp.float32),
    )(parts, bias2d)


def kernel(x, edge_index, fouriercoeffs, bias):
    n, in_feats = x.shape
    out_feats = fouriercoeffs.shape[1]
    grid_size = fouriercoeffs.shape[3]
    e = edge_index.shape[1]
    assert e % CH == 0
    w2 = jnp.transpose(fouriercoeffs, (0, 3, 2, 1)).reshape(
        2 * grid_size * in_feats, out_feats).astype(jnp.bfloat16)
    msg = _fourier_msg(x, w2, grid_size)
    pidx3 = edge_index.reshape(2, e // CH, CH).transpose(1, 0, 2)
    row_win = n - (NS - 1) * ((n // NS) // 8 * 8)
    zeros_blk = jnp.zeros((row_win, out_feats), jnp.float32)
    parts = _edge_scatter(msg, pidx3, zeros_blk)
    return _combine(parts, bias.reshape(1, out_feats))


# speedup vs baseline: 10.4125x; 1.5871x over previous
import functools

import jax
import jax.numpy as jnp
from jax import lax
from jax.experimental import pallas as pl
from jax.experimental.pallas import tpu as pltpu
from jax.experimental.pallas import tpu_sc as plsc

NC = 2
NS = 16
CH = 128


def _fourier_msg(x, w2, grid_size):
    n, in_feats = x.shape
    two_gi = w2.shape[0]
    out_feats = w2.shape[1]
    bn = 400
    assert n % bn == 0

    def body(x_ref, w_ref, o_ref, feats_ref):
        xb = x_ref[...]
        c1 = jnp.cos(xb)
        s1 = jnp.sin(xb)
        ck, sk = c1, s1
        for k in range(grid_size):
            feats_ref[:, k * in_feats:(k + 1) * in_feats] = ck.astype(jnp.bfloat16)
            feats_ref[:, (grid_size + k) * in_feats:(grid_size + k + 1) * in_feats] = (
                sk.astype(jnp.bfloat16))
            if k + 1 < grid_size:
                ck, sk = ck * c1 - sk * s1, sk * c1 + ck * s1
        o_ref[...] = jnp.dot(feats_ref[...], w_ref[...],
                             preferred_element_type=jnp.float32)

    return pl.pallas_call(
        body,
        grid=(n // bn,),
        in_specs=[
            pl.BlockSpec((bn, in_feats), lambda i: (i, 0)),
            pl.BlockSpec((two_gi, out_feats), lambda i: (0, 0)),
        ],
        out_specs=pl.BlockSpec((bn, out_feats), lambda i: (i, 0)),
        out_shape=jax.ShapeDtypeStruct((n, out_feats), jnp.float32),
        scratch_shapes=[pltpu.VMEM((bn, two_gi), jnp.bfloat16)],
    )(x, w2)


def _edge_scatter(msg, pidx3, zeros_blk):
    n, out_feats = msg.shape
    nchunks = pidx3.shape[0]
    nw = NC * NS
    full_rounds = nchunks // nw
    extra = nchunks - full_rounds * nw
    npairs = full_rounds // 2
    assert full_rounds % 2 == 0
    row_stride = (n // NS) // 8 * 8
    row_win = n - (NS - 1) * row_stride
    assert row_win >= row_stride and row_win % 8 == 0

    mesh = plsc.VectorSubcoreMesh(core_axis_name="c", subcore_axis_name="s")

    @functools.partial(
        pl.kernel,
        out_type=jax.ShapeDtypeStruct((NC, n, out_feats), jnp.float32),
        mesh=mesh,
        scratch_types=[
            pltpu.VMEM((2, CH), jnp.int32),
            pltpu.VMEM((2, CH), jnp.int32),
            pltpu.VMEM((CH, out_feats), jnp.float32),
            pltpu.VMEM((CH, out_feats), jnp.float32),
            pltpu.VMEM_SHARED((n, out_feats), jnp.float32),
            pltpu.SemaphoreType.DMA,
            pltpu.SemaphoreType.DMA,
        ],
    )
    def k(msg_hbm, p3_hbm, zero_hbm, out_hbm,
          pidx0, pidx1, rows0, rows1, acc, sem0, sem1):
        c = lax.axis_index("c")
        s = lax.axis_index("s")
        w = c * NS + s

        pltpu.sync_copy(zero_hbm, acc.at[pl.ds(s * row_stride, row_win)])
        plsc.subcore_barrier()

        pltpu.sync_copy(p3_hbm.at[w], pidx0)
        pltpu.async_copy(msg_hbm.at[pidx0.at[0]], rows0, sem0)

        @pl.loop(0, npairs)
        def _(p):
            ga = w + nw * (2 * p)
            gb = ga + nw
            pltpu.sync_copy(p3_hbm.at[gb], pidx1)
            cb = pltpu.async_copy(msg_hbm.at[pidx1.at[0]], rows1, sem1)
            pltpu.make_async_copy(msg_hbm.at[pidx0.at[0]], rows0, sem0).wait()
            pltpu.sync_copy(rows0, acc.at[pidx0.at[1]], add=True)
            @pl.when(p < npairs - 1)
            def _():
                pltpu.sync_copy(p3_hbm.at[gb + nw], pidx0)
                pltpu.async_copy(msg_hbm.at[pidx0.at[0]], rows0, sem0)
            cb.wait()
            pltpu.sync_copy(rows1, acc.at[pidx1.at[1]], add=True)

        @pl.when(w < extra)
        def _():
            pltpu.sync_copy(p3_hbm.at[full_rounds * nw + w], pidx0)
            pltpu.async_copy(msg_hbm.at[pidx0.at[0]], rows0, sem0).wait()
            pltpu.sync_copy(rows0, acc.at[pidx0.at[1]], add=True)

        plsc.subcore_barrier()
        pltpu.sync_copy(acc.at[pl.ds(s * row_stride, row_win)],
                        out_hbm.at[c].at[pl.ds(s * row_stride, row_win)])

    return k(msg, pidx3, zeros_blk)


def _combine(parts, bias2d):
    _, n, out_feats = parts.shape
    bn = 1000
    assert n % bn == 0

    def body(p_ref, b_ref, o_ref):
        o_ref[...] = p_ref[0] + p_ref[1] + b_ref[...]

    return pl.pallas_call(
        body,
        grid=(n // bn,),
        in_specs=[
            pl.BlockSpec((NC, bn, out_feats), lambda i: (0, i, 0)),
            pl.BlockSpec((1, out_feats), lambda i: (0, 0)),
        ],
        out_specs=pl.BlockSpec((bn, out_feats), lambda i: (i, 0)),
        out_shape=jax.ShapeDtypeStruct((n, out_feats), jnp.float32),
    )(parts, bias2d)


def kernel(x, edge_index, fouriercoeffs, bias):
    n, in_feats = x.shape
    out_feats = fouriercoeffs.shape[1]
    grid_size = fouriercoeffs.shape[3]
    e = edge_index.shape[1]
    assert e % CH == 0
    w2 = jnp.transpose(fouriercoeffs, (0, 3, 2, 1)).reshape(
        2 * grid_size * in_feats, out_feats).astype(jnp.bfloat16)
    msg = _fourier_msg(x, w2, grid_size)
    pidx3 = edge_index.reshape(2, e // CH, CH).transpose(1, 0, 2)
    row_win = n - (NS - 1) * ((n // NS) // 8 * 8)
    zeros_blk = jnp.zeros((row_win, out_feats), jnp.float32)
    parts = _edge_scatter(msg, pidx3, zeros_blk)
    return _combine(parts, bias.reshape(1, out_feats))


# speedup vs baseline: 11.2347x; 1.0790x over previous
import functools

import jax
import jax.numpy as jnp
from jax import lax
from jax.experimental import pallas as pl
from jax.experimental.pallas import tpu as pltpu
from jax.experimental.pallas import tpu_sc as plsc

NC = 2
NS = 16
CH = 128


def _fourier_msg(x, w2, grid_size):
    n, in_feats = x.shape
    two_gi = w2.shape[0]
    out_feats = w2.shape[1]
    bn = 400
    assert n % bn == 0

    def body(x_ref, w_ref, o_ref, feats_ref):
        xb = x_ref[...]
        c1 = jnp.cos(xb)
        s1 = jnp.sin(xb)
        ck, sk = c1, s1
        for k in range(grid_size):
            feats_ref[:, k * in_feats:(k + 1) * in_feats] = ck.astype(jnp.bfloat16)
            feats_ref[:, (grid_size + k) * in_feats:(grid_size + k + 1) * in_feats] = (
                sk.astype(jnp.bfloat16))
            if k + 1 < grid_size:
                ck, sk = ck * c1 - sk * s1, sk * c1 + ck * s1
        o_ref[...] = jnp.dot(feats_ref[...], w_ref[...],
                             preferred_element_type=jnp.float32)

    return pl.pallas_call(
        body,
        grid=(n // bn,),
        in_specs=[
            pl.BlockSpec((bn, in_feats), lambda i: (i, 0)),
            pl.BlockSpec((two_gi, out_feats), lambda i: (0, 0)),
        ],
        out_specs=pl.BlockSpec((bn, out_feats), lambda i: (i, 0)),
        out_shape=jax.ShapeDtypeStruct((n, out_feats), jnp.float32),
        scratch_shapes=[pltpu.VMEM((bn, two_gi), jnp.bfloat16)],
    )(x, w2)


def _edge_scatter(msg, pidx3, zeros_blk):
    n, out_feats = msg.shape
    nchunks = pidx3.shape[0]
    nw = NC * NS
    full_rounds = nchunks // nw
    extra = nchunks - full_rounds * nw
    npairs = full_rounds // 2
    assert full_rounds % 2 == 0
    row_stride = (n // NS) // 8 * 8
    row_win = n - (NS - 1) * row_stride
    assert row_win >= row_stride and row_win % 8 == 0

    mesh = plsc.VectorSubcoreMesh(core_axis_name="c", subcore_axis_name="s")

    blk = 26
    nblk = full_rounds // blk
    assert full_rounds == nblk * blk and blk % 2 == 0

    @functools.partial(
        pl.kernel,
        out_type=jax.ShapeDtypeStruct((NC, n, out_feats), jnp.float32),
        mesh=mesh,
        scratch_types=[
            pltpu.VMEM((blk, 2, CH), jnp.int32),
            pltpu.VMEM((blk, 2, CH), jnp.int32),
            pltpu.VMEM((2, CH), jnp.int32),
            pltpu.VMEM((CH, out_feats), jnp.float32),
            pltpu.VMEM((CH, out_feats), jnp.float32),
            pltpu.VMEM_SHARED((n, out_feats), jnp.float32),
            pltpu.SemaphoreType.DMA,
            pltpu.SemaphoreType.DMA,
            pltpu.SemaphoreType.DMA,
            pltpu.SemaphoreType.DMA,
        ],
    )
    def k(msg_hbm, p3_hbm, zero_hbm, out_hbm,
          pb0, pb1, lft, rows0, rows1, acc, semi0, semi1, sem0, sem1):
        c = lax.axis_index("c")
        s = lax.axis_index("s")
        w = c * NS + s
        base = w * full_rounds

        pbufs = (pb0, pb1)
        isems = (semi0, semi1)
        idesc = pltpu.async_copy(p3_hbm.at[pl.ds(base, blk)], pb0, semi0)
        pltpu.sync_copy(zero_hbm, acc.at[pl.ds(s * row_stride, row_win)])
        plsc.subcore_barrier()

        for ib in range(nblk):
            pb = pbufs[ib % 2]
            idesc.wait()
            if ib + 1 < nblk:
                idesc = pltpu.async_copy(
                    p3_hbm.at[pl.ds(base + (ib + 1) * blk, blk)],
                    pbufs[(ib + 1) % 2], isems[(ib + 1) % 2])

            pltpu.async_copy(msg_hbm.at[pb.at[0].at[0]], rows0, sem0)

            @pl.loop(0, blk // 2)
            def _(p):
                a = 2 * p
                b = a + 1
                cb = pltpu.async_copy(msg_hbm.at[pb.at[b].at[0]], rows1, sem1)
                pltpu.make_async_copy(msg_hbm.at[pb.at[a].at[0]], rows0,
                                      sem0).wait()
                pltpu.sync_copy(rows0, acc.at[pb.at[a].at[1]], add=True)
                @pl.when(p < blk // 2 - 1)
                def _():
                    pltpu.async_copy(msg_hbm.at[pb.at[a + 2].at[0]], rows0,
                                     sem0)
                cb.wait()
                pltpu.sync_copy(rows1, acc.at[pb.at[b].at[1]], add=True)

        @pl.when(w < extra)
        def _():
            pltpu.sync_copy(p3_hbm.at[full_rounds * nw + w], lft)
            pltpu.async_copy(msg_hbm.at[lft.at[0]], rows0, sem0).wait()
            pltpu.sync_copy(rows0, acc.at[lft.at[1]], add=True)

        plsc.subcore_barrier()
        pltpu.sync_copy(acc.at[pl.ds(s * row_stride, row_win)],
                        out_hbm.at[c].at[pl.ds(s * row_stride, row_win)])

    return k(msg, pidx3, zeros_blk)


def _combine(parts, bias2d):
    _, n, out_feats = parts.shape
    bn = 1000
    assert n % bn == 0

    def body(p_ref, b_ref, o_ref):
        o_ref[...] = p_ref[0] + p_ref[1] + b_ref[...]

    return pl.pallas_call(
        body,
        grid=(n // bn,),
        in_specs=[
            pl.BlockSpec((NC, bn, out_feats), lambda i: (0, i, 0)),
            pl.BlockSpec((1, out_feats), lambda i: (0, 0)),
        ],
        out_specs=pl.BlockSpec((bn, out_feats), lambda i: (i, 0)),
        out_shape=jax.ShapeDtypeStruct((n, out_feats), jnp.float32),
    )(parts, bias2d)


def kernel(x, edge_index, fouriercoeffs, bias):
    n, in_feats = x.shape
    out_feats = fouriercoeffs.shape[1]
    grid_size = fouriercoeffs.shape[3]
    e = edge_index.shape[1]
    assert e % CH == 0
    w2 = jnp.transpose(fouriercoeffs, (0, 3, 2, 1)).reshape(
        2 * grid_size * in_feats, out_feats).astype(jnp.bfloat16)
    msg = _fourier_msg(x, w2, grid_size)
    pidx3 = edge_index.reshape(2, e // CH, CH).transpose(1, 0, 2)
    row_win = n - (NS - 1) * ((n // NS) // 8 * 8)
    zeros_blk = jnp.zeros((row_win, out_feats), jnp.float32)
    parts = _edge_scatter(msg, pidx3, zeros_blk)
    return _combine(parts, bias.reshape(1, out_feats))
